# baseline (speedup 1.0000x reference)
import functools
import math

import numpy as np
import jax
import jax.numpy as jnp
from jax.experimental import pallas as pl
from jax.experimental.pallas import tpu as pltpu

_WINDOW_SIZE = 11
_SIGMA = 1.5
_C1 = 0.01 ** 2
_C2 = 0.03 ** 2


def _gauss_taps(window_size=_WINDOW_SIZE, sigma=_SIGMA):
    g = [math.exp(-((x - window_size // 2) ** 2) / float(2 * sigma ** 2))
         for x in range(window_size)]
    s = sum(g)
    return [v / s for v in g]


def _band_matrix(n, taps, p):
    m = np.zeros((n, n), dtype=np.float32)
    for d in range(-p, p + 1):
        m += np.float32(taps[d + p]) * np.eye(n, k=-d, dtype=np.float32)
    return m


def _ssim_block(x1_ref, x2_ref, bw_ref, bh_ref, out_ref, *, nslices, h):
    bw = bw_ref[...]
    bh = bh_ref[...]

    total = None
    for s in range(nslices):
        x1 = x1_ref[s * h:(s + 1) * h, :]
        x2 = x2_ref[s * h:(s + 1) * h, :]
        x1b = x1.astype(jnp.bfloat16)
        x2b = x2.astype(jnp.bfloat16)
        qb = (x1 * x1 + x2 * x2).astype(jnp.bfloat16)
        p12 = (x1 * x2).astype(jnp.bfloat16)

        xs = jnp.concatenate([x1b, x2b, qb, p12], axis=0)
        t = jnp.dot(xs, bw, preferred_element_type=jnp.float32)

        tl = jnp.concatenate([t[m * h:(m + 1) * h] for m in range(4)],
                             axis=1)
        u = jax.lax.dot_general(
            tl, bh, (((0,), (0,)), ((), ())),
            preferred_element_type=jnp.float32)

        mu1 = u[0:h]
        mu2 = u[h:2 * h]
        ee = u[2 * h:3 * h]
        e12 = u[3 * h:4 * h]

        c = mu1 * mu2
        ab = mu1 * mu1 + mu2 * mu2
        num4 = (c + _C1 * 0.5) * ((e12 - c) + _C2 * 0.5)
        den = (ab + _C1) * ((ee - ab) + _C2)
        part = jnp.sum(num4 / den)
        total = part if total is None else total + part
    out_ref[...] = jnp.broadcast_to(total, out_ref.shape)


def kernel(img1, img2):
    assert img1.shape == img2.shape
    n, c, h, w = img1.shape
    nc = n * c
    p = _WINDOW_SIZE // 2
    taps = _gauss_taps()
    bw = jnp.asarray(_band_matrix(w, taps, p), dtype=jnp.bfloat16)
    bh = jnp.asarray(_band_matrix(h, taps, p), dtype=jnp.float32)

    x1 = img1.reshape(nc * h, w)
    x2 = img2.reshape(nc * h, w)

    nslices = 1
    for b in range(1, nc + 1):
        if nc % b == 0 and b * h * w * 4 <= 4 * 1024 * 1024:
            nslices = b
    steps = nc // nslices

    body = functools.partial(_ssim_block, nslices=nslices, h=h)

    out = pl.pallas_call(
        body,
        out_shape=jax.ShapeDtypeStruct((steps, 8, 128), jnp.float32),
        grid=(steps,),
        in_specs=[
            pl.BlockSpec((nslices * h, w), lambda i: (i, 0)),
            pl.BlockSpec((nslices * h, w), lambda i: (i, 0)),
            pl.BlockSpec((w, w), lambda i: (0, 0)),
            pl.BlockSpec((h, h), lambda i: (0, 0)),
        ],
        out_specs=pl.BlockSpec((1, 8, 128), lambda i: (i, 0, 0)),
        compiler_params=pltpu.CompilerParams(
            dimension_semantics=("parallel",),
            vmem_limit_bytes=64 * 1024 * 1024,
        ),
    )(x1, x2, bw, bh)

    return out[:, 0, 0].sum() * (4.0 / jnp.float32(nc * h * w))


# speedup vs baseline: 1.1025x; 1.1025x over previous
import functools
import math

import numpy as np
import jax
import jax.numpy as jnp
from jax.experimental import pallas as pl
from jax.experimental.pallas import tpu as pltpu

_WINDOW_SIZE = 11
_SIGMA = 1.5
_C1 = 0.01 ** 2
_C2 = 0.03 ** 2


def _gauss_taps(window_size=_WINDOW_SIZE, sigma=_SIGMA):
    g = [math.exp(-((x - window_size // 2) ** 2) / float(2 * sigma ** 2))
         for x in range(window_size)]
    s = sum(g)
    return [v / s for v in g]


def _band_matrix(n, taps, p):
    m = np.zeros((n, n), dtype=np.float32)
    for d in range(-p, p + 1):
        m += np.float32(taps[d + p]) * np.eye(n, k=-d, dtype=np.float32)
    return m


def _ssim_block(x1_ref, x2_ref, bw_ref, bh_ref, out_ref, *, nslices, h):
    bw = bw_ref[...]
    bh = bh_ref[...]

    total = None
    for s in range(nslices):
        x1 = x1_ref[s * h:(s + 1) * h, :]
        x2 = x2_ref[s * h:(s + 1) * h, :]
        x1b = x1.astype(jnp.bfloat16)
        x2b = x2.astype(jnp.bfloat16)
        qb = (x1 * x1 + x2 * x2).astype(jnp.bfloat16)
        p12 = (x1 * x2).astype(jnp.bfloat16)

        xs = jnp.concatenate([x1b, x2b, qb, p12], axis=0)
        t = jnp.dot(xs, bw, preferred_element_type=jnp.float32)
        tb = t.astype(jnp.bfloat16)

        tl = jnp.concatenate([tb[m * h:(m + 1) * h] for m in range(4)],
                             axis=1)
        u = jax.lax.dot_general(
            tl, bh, (((0,), (0,)), ((), ())),
            preferred_element_type=jnp.float32)

        mu1 = u[0:h]
        mu2 = u[h:2 * h]
        ee = u[2 * h:3 * h]
        e12 = u[3 * h:4 * h]

        c = mu1 * mu2
        ab = mu1 * mu1 + mu2 * mu2
        num4 = (c + _C1 * 0.5) * ((e12 - c) + _C2 * 0.5)
        den = (ab + _C1) * ((ee - ab) + _C2)
        part = jnp.sum(num4 / den)
        total = part if total is None else total + part
    out_ref[...] = jnp.broadcast_to(total, out_ref.shape)


def kernel(img1, img2):
    assert img1.shape == img2.shape
    n, c, h, w = img1.shape
    nc = n * c
    p = _WINDOW_SIZE // 2
    taps = _gauss_taps()
    bw = jnp.asarray(_band_matrix(w, taps, p), dtype=jnp.bfloat16)
    bh = jnp.asarray(_band_matrix(h, taps, p), dtype=jnp.bfloat16)

    x1 = img1.reshape(nc * h, w)
    x2 = img2.reshape(nc * h, w)

    nslices = 1
    for b in range(1, nc + 1):
        if nc % b == 0 and b * h * w * 4 <= 2 * 1024 * 1024:
            nslices = b
    steps = nc // nslices

    body = functools.partial(_ssim_block, nslices=nslices, h=h)

    out = pl.pallas_call(
        body,
        out_shape=jax.ShapeDtypeStruct((steps, 8, 128), jnp.float32),
        grid=(steps,),
        in_specs=[
            pl.BlockSpec((nslices * h, w), lambda i: (i, 0)),
            pl.BlockSpec((nslices * h, w), lambda i: (i, 0)),
            pl.BlockSpec((w, w), lambda i: (0, 0)),
            pl.BlockSpec((h, h), lambda i: (0, 0)),
        ],
        out_specs=pl.BlockSpec((1, 8, 128), lambda i: (i, 0, 0)),
        compiler_params=pltpu.CompilerParams(
            dimension_semantics=("parallel",),
            vmem_limit_bytes=64 * 1024 * 1024,
        ),
    )(x1, x2, bw, bh)

    return out[:, 0, 0].sum() * (4.0 / jnp.float32(nc * h * w))


# speedup vs baseline: 1.1486x; 1.0418x over previous
import functools
import math

import numpy as np
import jax
import jax.numpy as jnp
from jax.experimental import pallas as pl
from jax.experimental.pallas import tpu as pltpu

_WINDOW_SIZE = 11
_SIGMA = 1.5
_C1 = 0.01 ** 2
_C2 = 0.03 ** 2


def _gauss_taps(window_size=_WINDOW_SIZE, sigma=_SIGMA):
    g = [math.exp(-((x - window_size // 2) ** 2) / float(2 * sigma ** 2))
         for x in range(window_size)]
    s = sum(g)
    return [v / s for v in g]


def _band_matrix(n, taps, p):
    m = np.zeros((n, n), dtype=np.float32)
    for d in range(-p, p + 1):
        m += np.float32(taps[d + p]) * np.eye(n, k=-d, dtype=np.float32)
    return m


def _ssim_block(x1_ref, x2_ref, bw_ref, bh_ref, out_ref, *, nslices, h):
    bw = bw_ref[...]
    bh = bh_ref[...]

    total = None
    for s in range(nslices):
        x1 = x1_ref[s * h:(s + 1) * h, :]
        x2 = x2_ref[s * h:(s + 1) * h, :]
        x1b = x1.astype(jnp.bfloat16)
        x2b = x2.astype(jnp.bfloat16)
        qb = (x1 * x1 + x2 * x2).astype(jnp.bfloat16)
        p12 = (x1 * x2).astype(jnp.bfloat16)

        xs = jnp.concatenate([x1b, x2b, qb, p12], axis=0)
        t = jnp.dot(xs, bw, preferred_element_type=jnp.float32)
        tb = t.astype(jnp.bfloat16)

        tl = jnp.concatenate([tb[m * h:(m + 1) * h] for m in range(4)],
                             axis=1)
        u = jax.lax.dot_general(
            tl, bh, (((0,), (0,)), ((), ())),
            preferred_element_type=jnp.float32)

        mu1 = u[0:h]
        mu2 = u[h:2 * h]
        ee = u[2 * h:3 * h]
        e12 = u[3 * h:4 * h]

        c = mu1 * mu2
        ab = mu1 * mu1 + mu2 * mu2
        num4 = (c + _C1 * 0.5) * ((e12 - c) + _C2 * 0.5)
        den = (ab + _C1) * ((ee - ab) + _C2)
        part = jnp.sum(num4 / den)
        total = part if total is None else total + part
    out_ref[...] = jnp.broadcast_to(total, out_ref.shape)


def kernel(img1, img2):
    assert img1.shape == img2.shape
    n, c, h, w = img1.shape
    nc = n * c
    p = _WINDOW_SIZE // 2
    taps = _gauss_taps()
    bw = jnp.asarray(_band_matrix(w, taps, p), dtype=jnp.bfloat16)
    bh = jnp.asarray(_band_matrix(h, taps, p), dtype=jnp.bfloat16)

    x1 = img1.reshape(nc * h, w)
    x2 = img2.reshape(nc * h, w)

    nslices = 1
    for b in range(1, nc + 1):
        if nc % b == 0 and b * h * w * 4 <= 6 * 1024 * 1024:
            nslices = b
    steps = nc // nslices

    body = functools.partial(_ssim_block, nslices=nslices, h=h)

    out = pl.pallas_call(
        body,
        out_shape=jax.ShapeDtypeStruct((steps, 8, 128), jnp.float32),
        grid=(steps,),
        in_specs=[
            pl.BlockSpec((nslices * h, w), lambda i: (i, 0)),
            pl.BlockSpec((nslices * h, w), lambda i: (i, 0)),
            pl.BlockSpec((w, w), lambda i: (0, 0)),
            pl.BlockSpec((h, h), lambda i: (0, 0)),
        ],
        out_specs=pl.BlockSpec((1, 8, 128), lambda i: (i, 0, 0)),
        compiler_params=pltpu.CompilerParams(
            dimension_semantics=("parallel",),
            vmem_limit_bytes=64 * 1024 * 1024,
        ),
    )(x1, x2, bw, bh)

    return out[:, 0, 0].sum() * (4.0 / jnp.float32(nc * h * w))


# speedup vs baseline: 1.1520x; 1.0029x over previous
import functools
import math

import numpy as np
import jax
import jax.numpy as jnp
from jax.experimental import pallas as pl
from jax.experimental.pallas import tpu as pltpu

_WINDOW_SIZE = 11
_SIGMA = 1.5
_C1 = 0.01 ** 2
_C2 = 0.03 ** 2


def _gauss_taps(window_size=_WINDOW_SIZE, sigma=_SIGMA):
    g = [math.exp(-((x - window_size // 2) ** 2) / float(2 * sigma ** 2))
         for x in range(window_size)]
    s = sum(g)
    return [v / s for v in g]


def _band_matrix(n, taps, p):
    m = np.zeros((n, n), dtype=np.float32)
    for d in range(-p, p + 1):
        m += np.float32(taps[d + p]) * np.eye(n, k=-d, dtype=np.float32)
    return m


def _ssim_block(x1_ref, x2_ref, bw_ref, bh_ref, out_ref, *, nslices, h):
    bw = bw_ref[...]
    bh = bh_ref[...]

    total = None
    for s in range(nslices):
        x1 = x1_ref[s * h:(s + 1) * h, :]
        x2 = x2_ref[s * h:(s + 1) * h, :]
        x1b = x1.astype(jnp.bfloat16)
        x2b = x2.astype(jnp.bfloat16)
        qb = (x1 * x1 + x2 * x2).astype(jnp.bfloat16)
        p12 = (x1 * x2).astype(jnp.bfloat16)

        xs = jnp.concatenate([x1b, x2b, qb, p12], axis=0)
        t = jnp.dot(xs, bw, preferred_element_type=jnp.float32)
        tb = t.astype(jnp.bfloat16)

        tl = jnp.concatenate([tb[m * h:(m + 1) * h] for m in range(4)],
                             axis=1)
        u = jax.lax.dot_general(
            tl, bh, (((0,), (0,)), ((), ())),
            preferred_element_type=jnp.float32)

        mu1 = u[0:h]
        mu2 = u[h:2 * h]
        ee = u[2 * h:3 * h]
        e12 = u[3 * h:4 * h]

        c = mu1 * mu2
        ab = mu1 * mu1 + mu2 * mu2
        num4 = (c + _C1 * 0.5) * ((e12 - c) + _C2 * 0.5)
        den = (ab + _C1) * ((ee - ab) + _C2)
        part = jnp.sum(num4 / den)
        total = part if total is None else total + part
    out_ref[...] = jnp.broadcast_to(total, out_ref.shape)


def kernel(img1, img2):
    assert img1.shape == img2.shape
    n, c, h, w = img1.shape
    nc = n * c
    p = _WINDOW_SIZE // 2
    taps = _gauss_taps()
    bw = jnp.asarray(_band_matrix(w, taps, p), dtype=jnp.bfloat16)
    bh = jnp.asarray(_band_matrix(h, taps, p), dtype=jnp.bfloat16)

    x1 = img1.reshape(nc * h, w)
    x2 = img2.reshape(nc * h, w)

    nslices = 1
    for b in range(1, nc + 1):
        if nc % b == 0 and b * h * w * 4 <= 4 * 1024 * 1024:
            nslices = b
    steps = nc // nslices

    body = functools.partial(_ssim_block, nslices=nslices, h=h)

    out = pl.pallas_call(
        body,
        out_shape=jax.ShapeDtypeStruct((steps, 8, 128), jnp.float32),
        grid=(steps,),
        in_specs=[
            pl.BlockSpec((nslices * h, w), lambda i: (i, 0)),
            pl.BlockSpec((nslices * h, w), lambda i: (i, 0)),
            pl.BlockSpec((w, w), lambda i: (0, 0)),
            pl.BlockSpec((h, h), lambda i: (0, 0)),
        ],
        out_specs=pl.BlockSpec((1, 8, 128), lambda i: (i, 0, 0)),
        compiler_params=pltpu.CompilerParams(
            dimension_semantics=("arbitrary",),
            vmem_limit_bytes=64 * 1024 * 1024,
        ),
    )(x1, x2, bw, bh)

    return out[:, 0, 0].sum() * (4.0 / jnp.float32(nc * h * w))
